# baseline (speedup 1.0000x reference)
import functools

import jax
import jax.numpy as jnp
from jax import lax
from jax.experimental import pallas as pl
from jax.experimental.pallas import tpu as pltpu
from jax.experimental.pallas import tpu_sc as plsc

_THRESH = 0.7
_MIN_KEPT = 100000
_B = 4
_C = 19
_PB = 262144
_NW = 32
_NRB_TC = 30
_W = 512
_SC_ROWS = (32 - _NRB_TC) * 64
_ROW_OFF = 512 - _SC_ROWS
_WROWS = 8
_NCOLS = (_NW * _WROWS) // _SC_ROWS
_WCOLS = _W // _NCOLS
_LN2 = 0.6931471805599453


def _ce_math(x, t):
    mx = jnp.max(x, axis=0)
    shifted = x - mx[None]
    s_exp = jnp.sum(jnp.exp(shifted), axis=0)
    log_s = jnp.log(s_exp)
    cls = lax.broadcasted_iota(jnp.int32, x.shape, 0)
    x_t = jnp.sum(jnp.where(cls == t[None], shifted, 0.0), axis=0)
    logp_t = x_t - log_s
    return jnp.exp(logp_t), -logp_t


def _acc_8x128(a):
    r, w = a.shape
    acc = jnp.zeros((8, 128), jnp.float32)
    for i in range(r // 8):
        for j in range(w // 128):
            acc = acc + a[i * 8:(i + 1) * 8, j * 128:(j + 1) * 128]
    return acc


def _ce_block_kernel(x_ref, t_ref, s_ref, c_ref):
    pred, loss = _ce_math(x_ref[0], t_ref[0])

    m07 = pred < _THRESH
    s07 = _acc_8x128(jnp.where(m07, loss, 0.0))
    c07 = _acc_8x128(m07.astype(jnp.float32))

    first = pl.program_id(0) == 0

    @pl.when(first)
    def _():
        s_ref[...] = jnp.zeros_like(s_ref)
        c_ref[...] = jnp.zeros_like(c_ref)

    s_ref[...] += s07
    c_ref[...] += c07


def _log_f32(s):
    bits = lax.bitcast_convert_type(s, jnp.int32)
    e = lax.shift_right_arithmetic(bits, 23) - 127
    mbits = lax.bitwise_or(lax.bitwise_and(bits, jnp.int32(0x7FFFFF)),
                           jnp.int32(0x3F800000))
    m = lax.bitcast_convert_type(mbits, jnp.float32)
    r = (m - 1.0) / (m + 1.0)
    r2 = r * r
    p = jnp.float32(1.0 / 11.0)
    p = p * r2 + 1.0 / 9.0
    p = p * r2 + 1.0 / 7.0
    p = p * r2 + 1.0 / 5.0
    p = p * r2 + 1.0 / 3.0
    p = p * r2 + 1.0
    return e.astype(jnp.float32) * _LN2 + 2.0 * r * p


def _sc_ce_body(x_hbm, t_hbm, s_out, c_out,
                x_v, t_v, stage_v):
    wid = lax.axis_index("s") * 2 + lax.axis_index("c")
    row0 = _ROW_OFF + (wid // _NCOLS) * _WROWS
    col0 = (wid % _NCOLS) * _WCOLS
    pltpu.sync_copy(
        t_hbm.at[_B - 1, pl.ds(row0, _WROWS), pl.ds(col0, _WCOLS)], t_v)
    pltpu.sync_copy(
        x_hbm.at[_B - 1, :, pl.ds(row0, _WROWS), pl.ds(col0, _WCOLS)], x_v)

    z = jnp.zeros((16,), jnp.float32)
    carry = (z, z)
    for r in range(_WROWS):

        def grp_body(j, carry, r=r):
            s07, c07 = carry
            sl = pl.ds(j * 16, 16)
            t16 = t_v[r, sl]
            mx = x_v[0, r, sl]
            for c in range(1, _C):
                mx = jnp.maximum(mx, x_v[c, r, sl])
            se = jnp.zeros((16,), jnp.float32)
            xt = jnp.zeros((16,), jnp.float32)
            for c in range(_C):
                v = x_v[c, r, sl]
                se = se + jnp.exp(v - mx)
                xt = jnp.where(t16 == c, v, xt)
            log_s = _log_f32(se)
            logp_t = (xt - mx) - log_s
            loss = -logp_t
            pred = jnp.exp(logp_t)
            m = pred < _THRESH
            s07 = s07 + jnp.where(m, loss, 0.0)
            c07 = c07 + jnp.where(m, 1.0, 0.0)
            return s07, c07

        carry = plsc.parallel_loop(0, _WCOLS // 16, carry=carry,
                                   unroll=2)(grp_body)
    s07, c07 = carry
    stage_v[...] = s07
    pltpu.sync_copy(stage_v, s_out.at[wid])
    stage_v[...] = c07
    pltpu.sync_copy(stage_v, c_out.at[wid])


@functools.cache
def _sc_ce_call():
    return pl.kernel(
        _sc_ce_body,
        mesh=plsc.VectorSubcoreMesh(core_axis_name="c", subcore_axis_name="s"),
        out_type=[
            jax.ShapeDtypeStruct((_NW, 16), jnp.float32),
            jax.ShapeDtypeStruct((_NW, 16), jnp.float32),
        ],
        scratch_types=[
            pltpu.VMEM((_C, _WROWS, _WCOLS), jnp.float32),
            pltpu.VMEM((_WROWS, _WCOLS), jnp.int32),
            pltpu.VMEM((16,), jnp.float32),
        ],
    )


def _ce_full_kernel(x_ref, t_ref, pred_ref, loss_ref):
    pred, loss = _ce_math(x_ref[0], t_ref[0])
    pred_ref[0] = pred
    loss_ref[0] = loss


def _select_kernel(pred_ref, loss_ref, s_ref, c_ref):
    p = pred_ref[...]
    bits = lax.bitcast_convert_type(p, jnp.int32)
    k1 = jnp.int32(_MIN_KEPT + 1)

    def body(_, lohi):
        lo, hi = lohi
        mid = (lo + hi) >> 1
        cnt = jnp.sum((bits <= mid).astype(jnp.int32))
        ok = cnt >= k1
        return jnp.where(ok, lo, mid), jnp.where(ok, mid, hi)

    lo0 = jnp.int32(-1)
    hi0 = jnp.int32(0x3F800000)
    _, hi = lax.fori_loop(0, 31, body, (lo0, hi0))
    vk = lax.bitcast_convert_type(hi, jnp.float32)
    thresh = jnp.maximum(vk, jnp.float32(_THRESH))
    keep = p < thresh
    s = jnp.sum(jnp.where(keep, loss_ref[...], 0.0))
    c = jnp.sum(keep.astype(jnp.float32))
    s_ref[...] = jnp.full(s_ref.shape, s, jnp.float32)
    c_ref[...] = jnp.full(c_ref.shape, c, jnp.float32)


def kernel(outputs, target):
    B, C, H, W = outputs.shape
    R = 64
    GR = H // R
    nblk = _NRB_TC

    s_tc, c_tc = pl.pallas_call(
        _ce_block_kernel,
        grid=(nblk,),
        in_specs=[
            pl.BlockSpec((1, C, R, W), lambda g: (g // GR, 0, g % GR, 0)),
            pl.BlockSpec((1, R, W), lambda g: (g // GR, g % GR, 0)),
        ],
        out_specs=[
            pl.BlockSpec((8, 128), lambda g: (0, 0)),
            pl.BlockSpec((8, 128), lambda g: (0, 0)),
        ],
        out_shape=[
            jax.ShapeDtypeStruct((8, 128), jnp.float32),
            jax.ShapeDtypeStruct((8, 128), jnp.float32),
        ],
    )(outputs, target)

    sp, cp = _sc_ce_call()(outputs, target)

    s07v = jnp.sum(s_tc) + jnp.sum(sp)
    c07v = jnp.sum(c_tc) + jnp.sum(cp)

    def common(outs, tgt, s, c):
        return s, c

    def fallback(outs, tgt, s, c):
        nrb = (B * H) // R
        pred, loss = pl.pallas_call(
            _ce_full_kernel,
            grid=(nrb,),
            in_specs=[
                pl.BlockSpec((1, C, R, W), lambda g: (g // GR, 0, g % GR, 0)),
                pl.BlockSpec((1, R, W), lambda g: (g // GR, g % GR, 0)),
            ],
            out_specs=[
                pl.BlockSpec((1, R, W), lambda g: (g, 0, 0)),
                pl.BlockSpec((1, R, W), lambda g: (g, 0, 0)),
            ],
            out_shape=[
                jax.ShapeDtypeStruct((nrb, R, W), jnp.float32),
                jax.ShapeDtypeStruct((nrb, R, W), jnp.float32),
            ],
        )(outs, tgt)
        sf, cf = pl.pallas_call(
            _select_kernel,
            in_specs=[
                pl.BlockSpec(pred.shape, lambda: (0, 0, 0)),
                pl.BlockSpec(loss.shape, lambda: (0, 0, 0)),
            ],
            out_specs=[
                pl.BlockSpec((8, 128), lambda: (0, 0)),
                pl.BlockSpec((8, 128), lambda: (0, 0)),
            ],
            out_shape=[
                jax.ShapeDtypeStruct((8, 128), jnp.float32),
                jax.ShapeDtypeStruct((8, 128), jnp.float32),
            ],
        )(pred, loss)
        return sf[0, 0], cf[0, 0]

    s, c = lax.cond(
        c07v >= jnp.float32(_MIN_KEPT + 1), common, fallback,
        outputs, target, s07v, c07v,
    )
    return s / jnp.maximum(c, 1.0)


# speedup vs baseline: 1.0276x; 1.0276x over previous
import functools

import jax
import jax.numpy as jnp
from jax import lax
from jax.experimental import pallas as pl
from jax.experimental.pallas import tpu as pltpu
from jax.experimental.pallas import tpu_sc as plsc

_THRESH = 0.7
_MIN_KEPT = 100000
_B = 4
_C = 19
_PB = 262144
_NW = 32
_NRB_TC = 28
_W = 512
_SC_ROWS = (32 - _NRB_TC) * 64
_ROW_OFF = 512 - _SC_ROWS
_WROWS = 8
_NCOLS = (_NW * _WROWS) // _SC_ROWS
_WCOLS = _W // _NCOLS
_LN2 = 0.6931471805599453


def _ce_math(x, t):
    mx = jnp.max(x, axis=0)
    shifted = x - mx[None]
    s_exp = jnp.sum(jnp.exp(shifted), axis=0)
    log_s = jnp.log(s_exp)
    cls = lax.broadcasted_iota(jnp.int32, x.shape, 0)
    x_t = jnp.sum(jnp.where(cls == t[None], shifted, 0.0), axis=0)
    logp_t = x_t - log_s
    return jnp.exp(logp_t), -logp_t


def _acc_8x128(a):
    r, w = a.shape
    acc = jnp.zeros((8, 128), jnp.float32)
    for i in range(r // 8):
        for j in range(w // 128):
            acc = acc + a[i * 8:(i + 1) * 8, j * 128:(j + 1) * 128]
    return acc


def _ce_block_kernel(x_ref, t_ref, s_ref, c_ref):
    pred, loss = _ce_math(x_ref[0], t_ref[0])

    m07 = pred < _THRESH
    s07 = _acc_8x128(jnp.where(m07, loss, 0.0))
    c07 = _acc_8x128(m07.astype(jnp.float32))

    first = pl.program_id(0) == 0

    @pl.when(first)
    def _():
        s_ref[...] = jnp.zeros_like(s_ref)
        c_ref[...] = jnp.zeros_like(c_ref)

    s_ref[...] += s07
    c_ref[...] += c07


def _log_f32(s):
    bits = lax.bitcast_convert_type(s, jnp.int32)
    e = lax.shift_right_arithmetic(bits, 23) - 127
    mbits = lax.bitwise_or(lax.bitwise_and(bits, jnp.int32(0x7FFFFF)),
                           jnp.int32(0x3F800000))
    m = lax.bitcast_convert_type(mbits, jnp.float32)
    r = (m - 1.0) / (m + 1.0)
    r2 = r * r
    p = jnp.float32(1.0 / 11.0)
    p = p * r2 + 1.0 / 9.0
    p = p * r2 + 1.0 / 7.0
    p = p * r2 + 1.0 / 5.0
    p = p * r2 + 1.0 / 3.0
    p = p * r2 + 1.0
    return e.astype(jnp.float32) * _LN2 + 2.0 * r * p


def _sc_ce_body(x_hbm, t_hbm, s_out, c_out,
                x_v, t_v, stage_v):
    wid = lax.axis_index("s") * 2 + lax.axis_index("c")
    row0 = _ROW_OFF + (wid // _NCOLS) * _WROWS
    col0 = (wid % _NCOLS) * _WCOLS
    pltpu.sync_copy(
        t_hbm.at[_B - 1, pl.ds(row0, _WROWS), pl.ds(col0, _WCOLS)], t_v)
    pltpu.sync_copy(
        x_hbm.at[_B - 1, :, pl.ds(row0, _WROWS), pl.ds(col0, _WCOLS)], x_v)

    z = jnp.zeros((16,), jnp.float32)
    carry = (z, z)
    for r in range(_WROWS):

        def grp_body(j, carry, r=r):
            s07, c07 = carry
            sl = pl.ds(j * 16, 16)
            t16 = t_v[r, sl]
            mx = x_v[0, r, sl]
            for c in range(1, _C):
                mx = jnp.maximum(mx, x_v[c, r, sl])
            se = jnp.zeros((16,), jnp.float32)
            xt = jnp.zeros((16,), jnp.float32)
            for c in range(_C):
                v = x_v[c, r, sl]
                se = se + jnp.exp(v - mx)
                xt = jnp.where(t16 == c, v, xt)
            log_s = _log_f32(se)
            logp_t = (xt - mx) - log_s
            loss = -logp_t
            pred = jnp.exp(logp_t)
            m = pred < _THRESH
            s07 = s07 + jnp.where(m, loss, 0.0)
            c07 = c07 + jnp.where(m, 1.0, 0.0)
            return s07, c07

        carry = plsc.parallel_loop(0, _WCOLS // 16, carry=carry,
                                   unroll=2)(grp_body)
    s07, c07 = carry
    stage_v[...] = s07
    pltpu.sync_copy(stage_v, s_out.at[wid])
    stage_v[...] = c07
    pltpu.sync_copy(stage_v, c_out.at[wid])


@functools.cache
def _sc_ce_call():
    return pl.kernel(
        _sc_ce_body,
        mesh=plsc.VectorSubcoreMesh(core_axis_name="c", subcore_axis_name="s"),
        out_type=[
            jax.ShapeDtypeStruct((_NW, 16), jnp.float32),
            jax.ShapeDtypeStruct((_NW, 16), jnp.float32),
        ],
        scratch_types=[
            pltpu.VMEM((_C, _WROWS, _WCOLS), jnp.float32),
            pltpu.VMEM((_WROWS, _WCOLS), jnp.int32),
            pltpu.VMEM((16,), jnp.float32),
        ],
    )


def _ce_full_kernel(x_ref, t_ref, pred_ref, loss_ref):
    pred, loss = _ce_math(x_ref[0], t_ref[0])
    pred_ref[0] = pred
    loss_ref[0] = loss


def _select_kernel(pred_ref, loss_ref, s_ref, c_ref):
    p = pred_ref[...]
    bits = lax.bitcast_convert_type(p, jnp.int32)
    k1 = jnp.int32(_MIN_KEPT + 1)

    def body(_, lohi):
        lo, hi = lohi
        mid = (lo + hi) >> 1
        cnt = jnp.sum((bits <= mid).astype(jnp.int32))
        ok = cnt >= k1
        return jnp.where(ok, lo, mid), jnp.where(ok, mid, hi)

    lo0 = jnp.int32(-1)
    hi0 = jnp.int32(0x3F800000)
    _, hi = lax.fori_loop(0, 31, body, (lo0, hi0))
    vk = lax.bitcast_convert_type(hi, jnp.float32)
    thresh = jnp.maximum(vk, jnp.float32(_THRESH))
    keep = p < thresh
    s = jnp.sum(jnp.where(keep, loss_ref[...], 0.0))
    c = jnp.sum(keep.astype(jnp.float32))
    s_ref[...] = jnp.full(s_ref.shape, s, jnp.float32)
    c_ref[...] = jnp.full(c_ref.shape, c, jnp.float32)


def kernel(outputs, target):
    B, C, H, W = outputs.shape
    R = 64
    GR = H // R
    nblk = _NRB_TC

    s_tc, c_tc = pl.pallas_call(
        _ce_block_kernel,
        grid=(nblk,),
        in_specs=[
            pl.BlockSpec((1, C, R, W), lambda g: (g // GR, 0, g % GR, 0)),
            pl.BlockSpec((1, R, W), lambda g: (g // GR, g % GR, 0)),
        ],
        out_specs=[
            pl.BlockSpec((8, 128), lambda g: (0, 0)),
            pl.BlockSpec((8, 128), lambda g: (0, 0)),
        ],
        out_shape=[
            jax.ShapeDtypeStruct((8, 128), jnp.float32),
            jax.ShapeDtypeStruct((8, 128), jnp.float32),
        ],
    )(outputs, target)

    sp, cp = _sc_ce_call()(outputs, target)

    s07v = jnp.sum(s_tc) + jnp.sum(sp)
    c07v = jnp.sum(c_tc) + jnp.sum(cp)

    def common(outs, tgt, s, c):
        return s, c

    def fallback(outs, tgt, s, c):
        nrb = (B * H) // R
        pred, loss = pl.pallas_call(
            _ce_full_kernel,
            grid=(nrb,),
            in_specs=[
                pl.BlockSpec((1, C, R, W), lambda g: (g // GR, 0, g % GR, 0)),
                pl.BlockSpec((1, R, W), lambda g: (g // GR, g % GR, 0)),
            ],
            out_specs=[
                pl.BlockSpec((1, R, W), lambda g: (g, 0, 0)),
                pl.BlockSpec((1, R, W), lambda g: (g, 0, 0)),
            ],
            out_shape=[
                jax.ShapeDtypeStruct((nrb, R, W), jnp.float32),
                jax.ShapeDtypeStruct((nrb, R, W), jnp.float32),
            ],
        )(outs, tgt)
        sf, cf = pl.pallas_call(
            _select_kernel,
            in_specs=[
                pl.BlockSpec(pred.shape, lambda: (0, 0, 0)),
                pl.BlockSpec(loss.shape, lambda: (0, 0, 0)),
            ],
            out_specs=[
                pl.BlockSpec((8, 128), lambda: (0, 0)),
                pl.BlockSpec((8, 128), lambda: (0, 0)),
            ],
            out_shape=[
                jax.ShapeDtypeStruct((8, 128), jnp.float32),
                jax.ShapeDtypeStruct((8, 128), jnp.float32),
            ],
        )(pred, loss)
        return sf[0, 0], cf[0, 0]

    s, c = lax.cond(
        c07v >= jnp.float32(_MIN_KEPT + 1), common, fallback,
        outputs, target, s07v, c07v,
    )
    return s / jnp.maximum(c, 1.0)
